# baseline (speedup 1.0000x reference)
import functools

import jax
import jax.numpy as jnp
from jax import lax
from jax.experimental import pallas as pl
from jax.experimental.pallas import tpu as pltpu
from jax.experimental.pallas import tpu_sc as plsc

D = 64
B_POS = 16384
B_NEG = 81920
P = B_POS + B_NEG
V = 2 * 100000 - 1

NC = 2
NS = 16
NW = NC * NS
LANES = 16

TW = 128
MAIN_BLKS = 48
EPI_BLKS = 26
MAIN_COLS = MAIN_BLKS * NW * TW
ALIGN_COLS = 1562 * TW
TAIL_W = TW
TAIL_C0 = ALIGN_COLS
V_PAD = TAIL_C0 + TAIL_W

B_PER_W = P // NW
CHUNK = 384
N_CHUNKS = B_PER_W // CHUNK


def _transpose_block(in_ref, out_ref, width):
    lane = lax.iota(jnp.int32, 16)

    def group(g, carry):
        for d in range(D):
            rid = (lane + d) & (D - 1)
            cid = g * LANES + lane
            x = plsc.load_gather(in_ref, [rid, cid])
            plsc.store_scatter(out_ref, [cid >> 1, ((cid & 1) << 6) + rid], x)
        return carry

    lax.fori_loop(0, width // LANES, group, 0)


def _sc_transpose_relay(ut_hbm, vt_hbm, ut_tail, vt_tail, u_out, v_out,
                        in_u, in_v, tr_u, tr_v, si_u, si_v, so_u, so_v):
    wid = lax.axis_index("s") * NC + lax.axis_index("c")

    def col0(j):
        return pl.multiple_of((j * NW + wid) * TW, TW)

    def issue_in(j):
        k = j % 2
        c0 = col0(j)
        pltpu.async_copy(ut_hbm.at[:, pl.ds(c0, TW)], in_u.at[k], si_u.at[k])
        pltpu.async_copy(vt_hbm.at[:, pl.ds(c0, TW)], in_v.at[k], si_v.at[k])

    def wait_in(j):
        k = j % 2
        pltpu.make_async_copy(ut_hbm.at[:, pl.ds(0, TW)], in_u.at[k], si_u.at[k]).wait()
        pltpu.make_async_copy(vt_hbm.at[:, pl.ds(0, TW)], in_v.at[k], si_v.at[k]).wait()

    def issue_out(j):
        k = j % 2
        r0 = pl.multiple_of(col0(j) // 2, TW // 2)
        pltpu.async_copy(tr_u.at[k], u_out.at[pl.ds(r0, TW // 2)], so_u.at[k])
        pltpu.async_copy(tr_v.at[k], v_out.at[pl.ds(r0, TW // 2)], so_v.at[k])

    def wait_out(j):
        k = j % 2
        pltpu.make_async_copy(tr_u.at[k], u_out.at[pl.ds(0, TW // 2)], so_u.at[k]).wait()
        pltpu.make_async_copy(tr_v.at[k], v_out.at[pl.ds(0, TW // 2)], so_v.at[k]).wait()

    issue_in(0)

    def pair_body(t, carry):
        for h in range(2):
            j = 2 * t + h

            @pl.when(j + 1 < MAIN_BLKS)
            def _(j=j, h=h):
                issue_in_k(j + 1, 1 - h)

            wait_in_k(h)

            @pl.when(j >= 2)
            def _(h=h):
                wait_out_k(h)

            _transpose_block(in_u.at[h], tr_u.at[h], TW)
            _transpose_block(in_v.at[h], tr_v.at[h], TW)
            issue_out_k(j, h)
        return carry

    def issue_in_k(j, k):
        c0 = col0(j)
        pltpu.async_copy(ut_hbm.at[:, pl.ds(c0, TW)], in_u.at[k], si_u.at[k])
        pltpu.async_copy(vt_hbm.at[:, pl.ds(c0, TW)], in_v.at[k], si_v.at[k])

    def wait_in_k(k):
        pltpu.make_async_copy(ut_hbm.at[:, pl.ds(0, TW)], in_u.at[k], si_u.at[k]).wait()
        pltpu.make_async_copy(vt_hbm.at[:, pl.ds(0, TW)], in_v.at[k], si_v.at[k]).wait()

    def issue_out_k(j, k):
        r0 = pl.multiple_of(col0(j) // 2, TW // 2)
        pltpu.async_copy(tr_u.at[k], u_out.at[pl.ds(r0, TW // 2)], so_u.at[k])
        pltpu.async_copy(tr_v.at[k], v_out.at[pl.ds(r0, TW // 2)], so_v.at[k])

    def wait_out_k(k):
        pltpu.make_async_copy(tr_u.at[k], u_out.at[pl.ds(0, TW // 2)], so_u.at[k]).wait()
        pltpu.make_async_copy(tr_v.at[k], v_out.at[pl.ds(0, TW // 2)], so_v.at[k]).wait()

    lax.fori_loop(0, MAIN_BLKS // 2, pair_body, 0)
    wait_out_k(0)
    wait_out_k(1)

    @pl.when(wid < EPI_BLKS)
    def _():
        c0 = pl.multiple_of(MAIN_COLS + wid * TW, TW)
        r0 = pl.multiple_of(c0 // 2, TW // 2)
        pltpu.sync_copy(ut_hbm.at[:, pl.ds(c0, TW)], in_u.at[0])
        pltpu.sync_copy(vt_hbm.at[:, pl.ds(c0, TW)], in_v.at[0])
        _transpose_block(in_u.at[0], tr_u.at[0], TW)
        _transpose_block(in_v.at[0], tr_v.at[0], TW)
        pltpu.sync_copy(tr_u.at[0], u_out.at[pl.ds(r0, TW // 2)])
        pltpu.sync_copy(tr_v.at[0], v_out.at[pl.ds(r0, TW // 2)])

    @pl.when(wid == NW - 1)
    def _():
        pltpu.sync_copy(ut_tail, in_u.at[0])
        pltpu.sync_copy(vt_tail, in_v.at[0])
        _transpose_block(in_u.at[0], tr_u.at[0], TW)
        _transpose_block(in_v.at[0], tr_v.at[0], TW)
        pltpu.sync_copy(tr_u.at[0], u_out.at[pl.ds(TAIL_C0 // 2, TW // 2)])
        pltpu.sync_copy(tr_v.at[0], v_out.at[pl.ds(TAIL_C0 // 2, TW // 2)])


def _sc_scores(u_hbm, v_hbm, iu_hbm, iv_hbm, out_hbm,
               idx_u, idx_v, rows_u, rows_v, sc_v,
               sem_iu, sem_iv, sem_u, sem_v, sem_out):
    wid = lax.axis_index("s") * NC + lax.axis_index("c")
    base = wid * B_PER_W

    def issue_idx(c):
        k = c % 2
        off = base + c * CHUNK
        pltpu.async_copy(iu_hbm.at[pl.ds(off, CHUNK)], idx_u.at[k], sem_iu.at[k])
        pltpu.async_copy(iv_hbm.at[pl.ds(off, CHUNK)], idx_v.at[k], sem_iv.at[k])

    def wait_idx(c):
        k = c % 2
        pltpu.make_async_copy(iu_hbm.at[pl.ds(0, CHUNK)], idx_u.at[k], sem_iu.at[k]).wait()
        pltpu.make_async_copy(iv_hbm.at[pl.ds(0, CHUNK)], idx_v.at[k], sem_iv.at[k]).wait()

    def issue_gather(c):
        k = c % 2
        pltpu.async_copy(u_hbm.at[idx_u.at[k]], rows_u.at[k], sem_u.at[k])
        pltpu.async_copy(v_hbm.at[idx_v.at[k]], rows_v.at[k], sem_v.at[k])

    def wait_gather(c):
        k = c % 2
        pltpu.make_async_copy(u_hbm.at[idx_u.at[k]], rows_u.at[k], sem_u.at[k]).wait()
        pltpu.make_async_copy(v_hbm.at[idx_v.at[k]], rows_v.at[k], sem_v.at[k]).wait()

    issue_idx(0)
    issue_idx(1)
    wait_idx(0)
    issue_gather(0)

    for c in range(N_CHUNKS):
        k = c % 2
        wait_gather(c)
        if c + 1 < N_CHUNKS:
            wait_idx(c + 1)
            issue_gather(c + 1)
        if c + 2 < N_CHUNKS:
            issue_idx(c + 2)

        ru = rows_u.at[k]
        rv = rows_v.at[k]

        def block_body(b, carry2, ru=ru, rv=rv, k=k):
            rids = b * LANES + lax.iota(jnp.int32, 16)
            lane = lax.iota(jnp.int32, 16)
            acc = jnp.zeros((LANES,), jnp.float32)
            for d in range(D):
                cid = (lane + d) & (D - 1)
                ul = plsc.load_gather(ru, [rids, cid])
                vl = plsc.load_gather(rv, [rids, cid])
                acc = acc + ul * vl
            sc_v[k, pl.ds(b * LANES, LANES)] = acc
            return carry2

        lax.fori_loop(0, CHUNK // LANES, block_body, 0)
        if c >= 2:
            pltpu.make_async_copy(
                sc_v.at[k], out_hbm.at[pl.ds(0, CHUNK)], sem_out.at[k]).wait()
        pltpu.async_copy(sc_v.at[k], out_hbm.at[pl.ds(base + c * CHUNK, CHUNK)],
                         sem_out.at[k])

    for c in (N_CHUNKS - 2, N_CHUNKS - 1):
        k = c % 2
        pltpu.make_async_copy(
            sc_v.at[k], out_hbm.at[pl.ds(0, CHUNK)], sem_out.at[k]).wait()


def _tc_loss_body(s_ref, o_ref):
    s = s_ref[...]
    row = lax.broadcasted_iota(jnp.int32, s.shape, 0)
    x = jnp.where(row < B_POS // 128, s, -s)
    ls = jnp.minimum(x, 0.0) - jnp.log1p(jnp.exp(-jnp.abs(x)))
    o_ref[0, 0] = -jnp.sum(ls)


def kernel(pos_u, pos_v, neg_u, neg_v, u_weight, v_weight):
    all_u = jnp.concatenate([pos_u, neg_u])
    all_v = jnp.concatenate([pos_v, neg_v])

    ut = u_weight.T
    vt = v_weight.T
    ut_tail = jnp.pad(lax.slice(ut, (0, TAIL_C0), (D, V)),
                      ((0, 0), (0, V_PAD - V)))
    vt_tail = jnp.pad(lax.slice(vt, (0, TAIL_C0), (D, V)),
                      ((0, 0), (0, V_PAD - V)))

    mesh = plsc.VectorSubcoreMesh(core_axis_name="c", subcore_axis_name="s")

    relay_fn = functools.partial(
        pl.kernel,
        out_type=(jax.ShapeDtypeStruct((V_PAD // 2, 128), jnp.float32),
                  jax.ShapeDtypeStruct((V_PAD // 2, 128), jnp.float32)),
        mesh=mesh,
        scratch_types=[
            pltpu.VMEM((2, D, TW), jnp.float32),
            pltpu.VMEM((2, D, TW), jnp.float32),
            pltpu.VMEM((2, TW // 2, 128), jnp.float32),
            pltpu.VMEM((2, TW // 2, 128), jnp.float32),
            pltpu.SemaphoreType.DMA((2,)),
            pltpu.SemaphoreType.DMA((2,)),
            pltpu.SemaphoreType.DMA((2,)),
            pltpu.SemaphoreType.DMA((2,)),
        ],
        compiler_params=pltpu.CompilerParams(
            needs_layout_passes=False, use_tc_tiling_on_sc=True),
    )(_sc_transpose_relay)
    u_lin, v_lin = relay_fn(ut, vt, ut_tail, vt_tail)
    u_lin = u_lin.reshape(V_PAD, D)
    v_lin = v_lin.reshape(V_PAD, D)

    sc_fn = functools.partial(
        pl.kernel,
        out_type=jax.ShapeDtypeStruct((P,), jnp.float32),
        mesh=mesh,
        scratch_types=[
            pltpu.VMEM((2, CHUNK), jnp.int32),
            pltpu.VMEM((2, CHUNK), jnp.int32),
            pltpu.VMEM((2, CHUNK, D), jnp.float32),
            pltpu.VMEM((2, CHUNK, D), jnp.float32),
            pltpu.VMEM((2, CHUNK), jnp.float32),
            pltpu.SemaphoreType.DMA((2,)),
            pltpu.SemaphoreType.DMA((2,)),
            pltpu.SemaphoreType.DMA((2,)),
            pltpu.SemaphoreType.DMA((2,)),
            pltpu.SemaphoreType.DMA((2,)),
        ],
        compiler_params=pltpu.CompilerParams(
            needs_layout_passes=False, use_tc_tiling_on_sc=False),
    )(_sc_scores)
    scores = sc_fn(u_lin, v_lin, all_u, all_v)

    loss = pl.pallas_call(
        _tc_loss_body,
        out_shape=jax.ShapeDtypeStruct((1, 1), jnp.float32),
        out_specs=pl.BlockSpec(memory_space=pltpu.SMEM),
    )(scores.reshape(P // 128, 128))
    return loss[0, 0]


# speedup vs baseline: 1.3327x; 1.3327x over previous
import functools

import jax
import jax.numpy as jnp
from jax import lax
from jax.experimental import pallas as pl
from jax.experimental.pallas import tpu as pltpu
from jax.experimental.pallas import tpu_sc as plsc

D = 64
B_POS = 16384
B_NEG = 81920
P = B_POS + B_NEG

NC = 2
NS = 16
NW = NC * NS
B_PER_W = P // NW
CHUNK = 384
N_CHUNKS = B_PER_W // CHUNK
LANES = 16


def _sc_scores(u_hbm, v_hbm, iu_hbm, iv_hbm, out_hbm,
               idx_u, idx_v, rows_u, rows_v, sc_v,
               sem_iu, sem_iv, sem_u, sem_v, sem_out):
    wid = lax.axis_index("s") * NC + lax.axis_index("c")
    base = wid * B_PER_W

    def issue_idx(c):
        k = c % 2
        off = base + c * CHUNK
        pltpu.async_copy(iu_hbm.at[pl.ds(off, CHUNK)], idx_u.at[k], sem_iu.at[k])
        pltpu.async_copy(iv_hbm.at[pl.ds(off, CHUNK)], idx_v.at[k], sem_iv.at[k])

    def wait_idx(c):
        k = c % 2
        pltpu.make_async_copy(iu_hbm.at[pl.ds(0, CHUNK)], idx_u.at[k], sem_iu.at[k]).wait()
        pltpu.make_async_copy(iv_hbm.at[pl.ds(0, CHUNK)], idx_v.at[k], sem_iv.at[k]).wait()

    def issue_gather(c):
        k = c % 2
        pltpu.async_copy(u_hbm.at[idx_u.at[k]], rows_u.at[k], sem_u.at[k])
        pltpu.async_copy(v_hbm.at[idx_v.at[k]], rows_v.at[k], sem_v.at[k])

    def wait_gather(c):
        k = c % 2
        pltpu.make_async_copy(u_hbm.at[idx_u.at[k]], rows_u.at[k], sem_u.at[k]).wait()
        pltpu.make_async_copy(v_hbm.at[idx_v.at[k]], rows_v.at[k], sem_v.at[k]).wait()

    issue_idx(0)
    issue_idx(1)
    wait_idx(0)
    issue_gather(0)

    for c in range(N_CHUNKS):
        k = c % 2
        wait_gather(c)
        if c + 1 < N_CHUNKS:
            wait_idx(c + 1)
            issue_gather(c + 1)
        if c + 2 < N_CHUNKS:
            issue_idx(c + 2)

        ru = rows_u.at[k]
        rv = rows_v.at[k]

        def block_body(b, carry2, ru=ru, rv=rv, k=k):
            rids = b * LANES + lax.iota(jnp.int32, 16)
            lane = lax.iota(jnp.int32, 16)
            acc = jnp.zeros((LANES,), jnp.float32)
            for d in range(D):
                cid = (lane + d) & (D - 1)
                ul = plsc.load_gather(ru, [rids, cid])
                vl = plsc.load_gather(rv, [rids, cid])
                acc = acc + ul * vl
            sc_v[k, pl.ds(b * LANES, LANES)] = acc
            return carry2

        lax.fori_loop(0, CHUNK // LANES, block_body, 0)
        if c >= 2:
            pltpu.make_async_copy(
                sc_v.at[k], out_hbm.at[pl.ds(0, CHUNK)], sem_out.at[k]).wait()
        pltpu.async_copy(sc_v.at[k], out_hbm.at[pl.ds(base + c * CHUNK, CHUNK)],
                         sem_out.at[k])

    for c in (N_CHUNKS - 2, N_CHUNKS - 1):
        k = c % 2
        pltpu.make_async_copy(
            sc_v.at[k], out_hbm.at[pl.ds(0, CHUNK)], sem_out.at[k]).wait()


def _tc_loss_body(s_ref, o_ref):
    s = s_ref[...]
    row = lax.broadcasted_iota(jnp.int32, s.shape, 0)
    x = jnp.where(row < B_POS // 128, s, -s)
    ls = jnp.minimum(x, 0.0) - jnp.log1p(jnp.exp(-jnp.abs(x)))
    o_ref[0, 0] = -jnp.sum(ls)


def kernel(pos_u, pos_v, neg_u, neg_v, u_weight, v_weight):
    all_u = jnp.concatenate([pos_u, neg_u])
    all_v = jnp.concatenate([pos_v, neg_v])

    mesh = plsc.VectorSubcoreMesh(core_axis_name="c", subcore_axis_name="s")
    sc_fn = functools.partial(
        pl.kernel,
        out_type=jax.ShapeDtypeStruct((P,), jnp.float32),
        mesh=mesh,
        scratch_types=[
            pltpu.VMEM((2, CHUNK), jnp.int32),
            pltpu.VMEM((2, CHUNK), jnp.int32),
            pltpu.VMEM((2, CHUNK, D), jnp.float32),
            pltpu.VMEM((2, CHUNK, D), jnp.float32),
            pltpu.VMEM((2, CHUNK), jnp.float32),
            pltpu.SemaphoreType.DMA((2,)),
            pltpu.SemaphoreType.DMA((2,)),
            pltpu.SemaphoreType.DMA((2,)),
            pltpu.SemaphoreType.DMA((2,)),
            pltpu.SemaphoreType.DMA((2,)),
        ],
        compiler_params=pltpu.CompilerParams(
            needs_layout_passes=False, use_tc_tiling_on_sc=False),
    )(_sc_scores)
    scores = sc_fn(u_weight, v_weight, all_u, all_v)

    loss = pl.pallas_call(
        _tc_loss_body,
        out_shape=jax.ShapeDtypeStruct((1, 1), jnp.float32),
        out_specs=pl.BlockSpec(memory_space=pltpu.SMEM),
    )(scores.reshape(P // 128, 128))
    return loss[0, 0]
